# Initial kernel scaffold; baseline (speedup 1.0000x reference)
import jax
import jax.numpy as jnp
from jax.experimental import pallas as pl


def kernel(location, category, defaultbox, targets):
    raise NotImplementedError("write your pallas kernel here")


# speedup vs baseline: 13.1021x; 13.1021x over previous
import functools

import jax
import jax.numpy as jnp
from jax import lax
from jax.experimental import pallas as pl
from jax.experimental.pallas import tpu as pltpu

NUM_CLASSES = 21
VAR0 = 0.1
VAR1 = 0.2
THRESHOLD = 0.5
NEGPOS_RATIO = 3


def _mbl_kernel(loc_ref, cat_ref, db_ref, tgt_ref, out_l_ref, out_c_ref,
                acc_ref, *, B, P, O, C):
    i = pl.program_id(0)

    tgt = tgt_ref[0]
    tx1 = tgt[:, 0:1]
    ty1 = tgt[:, 1:2]
    tx2 = tgt[:, 2:3]
    ty2 = tgt[:, 3:4]
    tlab = tgt[:, 4:5]

    db = db_ref[...]
    pcx = db[0:1, :]
    pcy = db[1:2, :]
    pw = db[2:3, :]
    ph = db[3:4, :]
    px1 = pcx - pw / 2.0
    py1 = pcy - ph / 2.0
    px2 = pcx + pw / 2.0
    py2 = pcy + ph / 2.0

    iw = jnp.clip(jnp.minimum(tx2, px2) - jnp.maximum(tx1, px1), 0.0, None)
    ih = jnp.clip(jnp.minimum(ty2, py2) - jnp.maximum(ty1, py1), 0.0, None)
    inter = iw * ih
    area_a = (tx2 - tx1) * (ty2 - ty1)
    area_b = (px2 - px1) * (py2 - py1)
    iou = inter / (area_a + area_b - inter)

    iota_p = lax.broadcasted_iota(jnp.int32, (1, P), 1)
    iota_t = lax.broadcasted_iota(jnp.int32, (O, 1), 0)

    row_max = jnp.max(iou, axis=1, keepdims=True)
    bpi = jnp.min(jnp.where(iou == row_max, iota_p, P), axis=1, keepdims=True)

    col_max = jnp.max(iou, axis=0, keepdims=True)
    bti = jnp.min(jnp.where(iou == col_max, iota_t, O), axis=0, keepdims=True)

    eq = bpi == iota_p
    last_t = jnp.max(jnp.where(eq, iota_t, -1), axis=0, keepdims=True)
    forced = last_t >= 0
    bti = jnp.where(forced, last_t, bti)
    bto = jnp.where(forced, 2.0, col_max)

    sel = (iota_t == bti).astype(jnp.float32)
    mx1 = jnp.sum(sel * tx1, axis=0, keepdims=True)
    my1 = jnp.sum(sel * ty1, axis=0, keepdims=True)
    mx2 = jnp.sum(sel * tx2, axis=0, keepdims=True)
    my2 = jnp.sum(sel * ty2, axis=0, keepdims=True)
    mlab = jnp.sum(sel * tlab, axis=0, keepdims=True)

    pos = jnp.logical_not(bto < THRESHOLD)
    posf = pos.astype(jnp.float32)

    g_cx = ((mx1 + mx2) / 2.0 - pcx) / (VAR0 * pw)
    g_cy = ((my1 + my2) / 2.0 - pcy) / (VAR0 * ph)
    g_w = jnp.log((mx2 - mx1) / pw) / VAR1
    g_h = jnp.log((my2 - my1) / ph) / VAR1

    locs = loc_ref[0]

    def _sl1(d):
        a = jnp.abs(d)
        return jnp.where(a < 1.0, 0.5 * d * d, a - 0.5)

    sl1 = (_sl1(locs[0:1, :] - g_cx) + _sl1(locs[1:2, :] - g_cy)
           + _sl1(locs[2:3, :] - g_w) + _sl1(locs[3:4, :] - g_h))
    loss_l_i = jnp.sum(sl1 * posf)

    cat = cat_ref[0]
    m = jnp.max(cat, axis=0, keepdims=True)
    s = jnp.sum(jnp.exp(cat - m), axis=0, keepdims=True)
    logz = jnp.log(s) + m
    conf = jnp.where(bto < THRESHOLD, 0, (mlab + 1.0).astype(jnp.int32))
    iota_c = lax.broadcasted_iota(jnp.int32, (C, 1), 0)
    gt_logit = jnp.sum(jnp.where(iota_c == conf, cat, 0.0), axis=0,
                       keepdims=True)
    ce_all = logz - gt_logit
    ce_mined = jnp.where(pos, 0.0, jnp.maximum(ce_all, 0.0))

    npos_i = jnp.sum(pos.astype(jnp.int32))
    k = jnp.minimum(NEGPOS_RATIO * npos_i, P - 1)

    vbits = lax.bitcast_convert_type(ce_mined, jnp.int32)

    def _bit_step(j, cand):
        test = cand | (1 << (30 - j))
        cnt = jnp.sum((vbits >= test).astype(jnp.int32))
        return jnp.where(cnt >= k, test, cand)

    tbits = lax.fori_loop(0, 31, _bit_step, jnp.int32(0))
    tval = lax.bitcast_convert_type(tbits, jnp.float32)
    gt_mask = vbits > tbits
    cnt_gt = jnp.sum(gt_mask.astype(jnp.int32))
    sum_gt = jnp.sum(jnp.where(gt_mask, ce_mined, 0.0))
    topk = sum_gt + (k - cnt_gt).astype(jnp.float32) * tval

    loss_c_i = jnp.sum(jnp.where(pos, ce_all, 0.0)) + topk

    @pl.when(i == 0)
    def _init():
        acc_ref[0] = 0.0
        acc_ref[1] = 0.0
        acc_ref[2] = 0.0

    acc_ref[0] += loss_l_i
    acc_ref[1] += loss_c_i
    acc_ref[2] += npos_i.astype(jnp.float32)

    @pl.when(i == B - 1)
    def _fin():
        n = acc_ref[2]
        out_l_ref[0, 0] = acc_ref[0] / n
        out_c_ref[0, 0] = acc_ref[1] / n


def kernel(location, category, defaultbox, targets):
    B, P, C = category.shape
    O = targets.shape[1]
    loc_t = jnp.transpose(location, (0, 2, 1))
    cat_t = jnp.transpose(category, (0, 2, 1))
    db_t = jnp.transpose(defaultbox, (1, 0))

    out_l, out_c = pl.pallas_call(
        functools.partial(_mbl_kernel, B=B, P=P, O=O, C=C),
        grid=(B,),
        in_specs=[
            pl.BlockSpec((1, 4, P), lambda i: (i, 0, 0)),
            pl.BlockSpec((1, C, P), lambda i: (i, 0, 0)),
            pl.BlockSpec((4, P), lambda i: (0, 0)),
            pl.BlockSpec((1, O, 5), lambda i: (i, 0, 0)),
        ],
        out_specs=[
            pl.BlockSpec(memory_space=pltpu.SMEM),
            pl.BlockSpec(memory_space=pltpu.SMEM),
        ],
        out_shape=[
            jax.ShapeDtypeStruct((1, 1), jnp.float32),
            jax.ShapeDtypeStruct((1, 1), jnp.float32),
        ],
        scratch_shapes=[pltpu.SMEM((3,), jnp.float32)],
    )(loc_t, cat_t, db_t, targets)
    return out_l[0, 0], out_c[0, 0]
